# Initial kernel scaffold; baseline (speedup 1.0000x reference)
import jax
import jax.numpy as jnp
from jax.experimental import pallas as pl


def kernel(x, edge_index, W1, b1, W2, b2):
    raise NotImplementedError("write your pallas kernel here")


# speedup vs baseline: 12.2577x; 12.2577x over previous
import functools

import jax
import jax.numpy as jnp
from jax import lax
from jax.experimental import pallas as pl
from jax.experimental.pallas import tpu as pltpu
from jax.experimental.pallas import tpu_sc as plsc

N_NODES = 10000
N_EDGES = 320000
FEAT_DIM = 128
NHID = 16
NUM_CLASS = 40

NC = 2
NS = 16
NW = NC * NS
E_PER_W = N_EDGES // NW
BATCH = 125
NB = E_PER_W // BATCH
N_PAD = 10240
ROWS_PER_TILE = N_PAD // NS


def _agg_body(sup_hbm, src_hbm, dst_hbm, zrow_hbm, out_hbm,
              src_v, dst_v, rows_v, drain_v, acc_sh):
    c = lax.axis_index("c")
    s = lax.axis_index("s")
    wid = c * NS + s
    pltpu.sync_copy(src_hbm.at[wid], src_v)
    pltpu.sync_copy(dst_hbm.at[wid], dst_v)
    pltpu.sync_copy(zrow_hbm, drain_v)
    pltpu.sync_copy(drain_v, acc_sh.at[pl.ds(s * ROWS_PER_TILE, ROWS_PER_TILE)])
    plsc.subcore_barrier()

    def body(j, carry):
        pltpu.sync_copy(sup_hbm.at[src_v.at[j]], rows_v)
        pltpu.sync_copy(rows_v, acc_sh.at[dst_v.at[j]], add=True)
        return carry

    lax.fori_loop(0, NB, body, 0)
    plsc.subcore_barrier()
    pltpu.sync_copy(acc_sh.at[pl.ds(s * ROWS_PER_TILE, ROWS_PER_TILE)], drain_v)
    pltpu.sync_copy(drain_v, out_hbm.at[c, pl.ds(s * ROWS_PER_TILE, ROWS_PER_TILE)])


_aggregate = functools.partial(
    pl.kernel,
    mesh=plsc.VectorSubcoreMesh(core_axis_name="c", subcore_axis_name="s"),
    out_type=jax.ShapeDtypeStruct((NC, N_PAD, NHID), jnp.float32),
    scratch_types=[
        pltpu.VMEM((NB, BATCH), jnp.int32),
        pltpu.VMEM((NB, BATCH), jnp.int32),
        pltpu.VMEM((BATCH, NHID), jnp.float32),
        pltpu.VMEM((ROWS_PER_TILE, NHID), jnp.float32),
        pltpu.VMEM_SHARED((N_PAD, NHID), jnp.float32),
    ],
    compiler_params=pltpu.CompilerParams(use_tc_tiling_on_sc=False),
)(_agg_body)


def _mm1_body(x_ref, w_ref, o_ref):
    o_ref[...] = jnp.dot(x_ref[...], w_ref[...],
                         preferred_element_type=jnp.float32)


def _relu_body(p_ref, b_ref, o_ref):
    o_ref[...] = jnp.maximum(p_ref[0] + p_ref[1] + b_ref[...], 0.0)


def _head_body(p_ref, w_ref, b_ref, o_ref):
    z = jnp.dot(p_ref[0, :N_NODES] + p_ref[1, :N_NODES], w_ref[...],
                preferred_element_type=jnp.float32) + b_ref[...]
    m = jnp.max(z, axis=1, keepdims=True)
    z = z - m
    lse = jnp.log(jnp.sum(jnp.exp(z), axis=1, keepdims=True))
    o_ref[...] = z - lse


def kernel(x, edge_index, W1, b1, W2, b2):
    src = edge_index[0].astype(jnp.int32).reshape(NW, NB, BATCH)
    dst = edge_index[1].astype(jnp.int32).reshape(NW, NB, BATCH)
    zrow = jnp.zeros((ROWS_PER_TILE, NHID), jnp.float32)

    support1 = pl.pallas_call(
        _mm1_body,
        out_shape=jax.ShapeDtypeStruct((N_NODES, NHID), jnp.float32),
    )(x, W1)

    part1 = _aggregate(support1, src, dst, zrow)

    h = pl.pallas_call(
        _relu_body,
        out_shape=jax.ShapeDtypeStruct((N_PAD, NHID), jnp.float32),
    )(part1, b1.reshape(1, NHID))

    part2 = _aggregate(h, src, dst, zrow)

    out = pl.pallas_call(
        _head_body,
        out_shape=jax.ShapeDtypeStruct((N_NODES, NUM_CLASS), jnp.float32),
    )(part2, W2, b2.reshape(1, NUM_CLASS))

    return out


# speedup vs baseline: 20.5316x; 1.6750x over previous
import functools

import jax
import jax.numpy as jnp
from jax import lax
from jax.experimental import pallas as pl
from jax.experimental.pallas import tpu as pltpu
from jax.experimental.pallas import tpu_sc as plsc

N_NODES = 10000
N_EDGES = 320000
FEAT_DIM = 128
NHID = 16
NUM_CLASS = 40

NC = 2
NS = 16
NW = NC * NS
E_PER_W = N_EDGES // NW
BATCH = 125
NB = E_PER_W // BATCH
N_PAD = 10240
ROWS_PER_TILE = N_PAD // NS


NBUF = 4


def _agg_body(sup_hbm, src_hbm, dst_hbm, zrow_hbm, out_hbm,
              src_v, dst_v, rows_v, drain_v, acc_sh, gsem):
    c = lax.axis_index("c")
    s = lax.axis_index("s")
    wid = c * NS + s
    pltpu.sync_copy(src_hbm.at[wid], src_v)
    pltpu.sync_copy(dst_hbm.at[wid], dst_v)
    pltpu.sync_copy(zrow_hbm, drain_v)
    pltpu.sync_copy(drain_v, acc_sh.at[pl.ds(s * ROWS_PER_TILE, ROWS_PER_TILE)])
    plsc.subcore_barrier()

    for b in range(NBUF):
        pltpu.async_copy(sup_hbm.at[src_v.at[b]], rows_v.at[b], gsem.at[b])

    def group(g, carry):
        for b in range(NBUF):
            j = g * NBUF + b
            pltpu.make_async_copy(sup_hbm.at[src_v.at[j]], rows_v.at[b],
                                  gsem.at[b]).wait()
            pltpu.sync_copy(rows_v.at[b], acc_sh.at[dst_v.at[j]], add=True)

            @pl.when(j + NBUF < NB)
            def _():
                pltpu.async_copy(sup_hbm.at[src_v.at[j + NBUF]], rows_v.at[b],
                                 gsem.at[b])
        return carry

    lax.fori_loop(0, NB // NBUF, group, 0)
    plsc.subcore_barrier()
    pltpu.sync_copy(acc_sh.at[pl.ds(s * ROWS_PER_TILE, ROWS_PER_TILE)], drain_v)
    pltpu.sync_copy(drain_v, out_hbm.at[c, pl.ds(s * ROWS_PER_TILE, ROWS_PER_TILE)])


_aggregate = functools.partial(
    pl.kernel,
    mesh=plsc.VectorSubcoreMesh(core_axis_name="c", subcore_axis_name="s"),
    out_type=jax.ShapeDtypeStruct((NC, N_PAD, NHID), jnp.float32),
    scratch_types=[
        pltpu.VMEM((NB, BATCH), jnp.int32),
        pltpu.VMEM((NB, BATCH), jnp.int32),
        pltpu.VMEM((NBUF, BATCH, NHID), jnp.float32),
        pltpu.VMEM((ROWS_PER_TILE, NHID), jnp.float32),
        pltpu.VMEM_SHARED((N_PAD, NHID), jnp.float32),
        pltpu.SemaphoreType.DMA((NBUF,)),
    ],
    compiler_params=pltpu.CompilerParams(use_tc_tiling_on_sc=False),
)(_agg_body)


def _mm1_body(x_ref, w_ref, o_ref):
    o_ref[...] = jnp.dot(x_ref[...], w_ref[...],
                         preferred_element_type=jnp.float32)


def _relu_body(p_ref, b_ref, o_ref):
    o_ref[...] = jnp.maximum(p_ref[0] + p_ref[1] + b_ref[...], 0.0)


def _head_body(p_ref, w_ref, b_ref, o_ref):
    z = jnp.dot(p_ref[0, :N_NODES] + p_ref[1, :N_NODES], w_ref[...],
                preferred_element_type=jnp.float32) + b_ref[...]
    m = jnp.max(z, axis=1, keepdims=True)
    z = z - m
    lse = jnp.log(jnp.sum(jnp.exp(z), axis=1, keepdims=True))
    o_ref[...] = z - lse


def kernel(x, edge_index, W1, b1, W2, b2):
    src = edge_index[0].astype(jnp.int32).reshape(NW, NB, BATCH)
    dst = edge_index[1].astype(jnp.int32).reshape(NW, NB, BATCH)
    zrow = jnp.zeros((ROWS_PER_TILE, NHID), jnp.float32)

    support1 = pl.pallas_call(
        _mm1_body,
        out_shape=jax.ShapeDtypeStruct((N_NODES, NHID), jnp.float32),
    )(x, W1)

    part1 = _aggregate(support1, src, dst, zrow)

    h = pl.pallas_call(
        _relu_body,
        out_shape=jax.ShapeDtypeStruct((N_PAD, NHID), jnp.float32),
    )(part1, b1.reshape(1, NHID))

    part2 = _aggregate(h, src, dst, zrow)

    out = pl.pallas_call(
        _head_body,
        out_shape=jax.ShapeDtypeStruct((N_NODES, NUM_CLASS), jnp.float32),
    )(part2, W2, b2.reshape(1, NUM_CLASS))

    return out


# speedup vs baseline: 21.0583x; 1.0257x over previous
import functools

import jax
import jax.numpy as jnp
from jax import lax
from jax.experimental import pallas as pl
from jax.experimental.pallas import tpu as pltpu
from jax.experimental.pallas import tpu_sc as plsc

N_NODES = 10000
N_EDGES = 320000
FEAT_DIM = 128
NHID = 16
NUM_CLASS = 40

NC = 2
NS = 16
NW = NC * NS
E_PER_W = N_EDGES // NW
BATCH = 125
NB = E_PER_W // BATCH
N_PAD = 10240
ROWS_PER_TILE = N_PAD // NS


NBUF = 8
HALF = NBUF // 2


def _agg_body(sup_hbm, src_hbm, dst_hbm, zrow_hbm, out_hbm,
              src_v, dst_v, rows_v, drain_v, acc_sh, gsem, ssem):
    c = lax.axis_index("c")
    s = lax.axis_index("s")
    wid = c * NS + s
    pltpu.sync_copy(src_hbm.at[wid], src_v)
    pltpu.sync_copy(dst_hbm.at[wid], dst_v)
    pltpu.sync_copy(zrow_hbm, drain_v)
    pltpu.sync_copy(drain_v, acc_sh.at[pl.ds(s * ROWS_PER_TILE, ROWS_PER_TILE)])
    plsc.subcore_barrier()

    def gather(j, b):
        pltpu.async_copy(sup_hbm.at[src_v.at[j]], rows_v.at[b], gsem.at[b])

    def gather_wait(j, b):
        pltpu.make_async_copy(sup_hbm.at[src_v.at[j]], rows_v.at[b],
                              gsem.at[b]).wait()

    def scat(j, b):
        pltpu.async_copy(rows_v.at[b], acc_sh.at[dst_v.at[j]], ssem.at[b],
                         add=True)

    def scat_wait(j, b):
        pltpu.make_async_copy(rows_v.at[b], acc_sh.at[dst_v.at[j]],
                              ssem.at[b]).wait()

    for b in range(HALF):
        gather(b, b)

    def group(g, carry):
        for b in range(NBUF):
            j = g * NBUF + b
            gather_wait(j, b)
            scat(j, b)
            b2 = (b + HALF) % NBUF
            jn = j + HALF
            jp = j - HALF

            @pl.when(jp >= 0)
            def _():
                scat_wait(jp, b2)

            @pl.when(jn < NB)
            def _():
                gather(jn, b2)
        return carry

    lax.fori_loop(0, NB // NBUF, group, 0)
    for b in range(HALF, NBUF):
        scat_wait(NB - NBUF + b, b)
    plsc.subcore_barrier()
    pltpu.sync_copy(acc_sh.at[pl.ds(s * ROWS_PER_TILE, ROWS_PER_TILE)], drain_v)
    pltpu.sync_copy(drain_v, out_hbm.at[c, pl.ds(s * ROWS_PER_TILE, ROWS_PER_TILE)])


_aggregate = functools.partial(
    pl.kernel,
    mesh=plsc.VectorSubcoreMesh(core_axis_name="c", subcore_axis_name="s"),
    out_type=jax.ShapeDtypeStruct((NC, N_PAD, NHID), jnp.float32),
    scratch_types=[
        pltpu.VMEM((NB, BATCH), jnp.int32),
        pltpu.VMEM((NB, BATCH), jnp.int32),
        pltpu.VMEM((NBUF, BATCH, NHID), jnp.float32),
        pltpu.VMEM((ROWS_PER_TILE, NHID), jnp.float32),
        pltpu.VMEM_SHARED((N_PAD, NHID), jnp.float32),
        pltpu.SemaphoreType.DMA((NBUF,)),
        pltpu.SemaphoreType.DMA((NBUF,)),
    ],
    compiler_params=pltpu.CompilerParams(use_tc_tiling_on_sc=False),
)(_agg_body)


def _mm1_body(x_ref, w_ref, o_ref):
    o_ref[...] = jnp.dot(x_ref[...], w_ref[...],
                         preferred_element_type=jnp.float32)


def _relu_body(p_ref, b_ref, o_ref):
    o_ref[...] = jnp.maximum(p_ref[0] + p_ref[1] + b_ref[...], 0.0)


def _head_body(p_ref, w_ref, b_ref, o_ref):
    z = jnp.dot(p_ref[0, :N_NODES] + p_ref[1, :N_NODES], w_ref[...],
                preferred_element_type=jnp.float32) + b_ref[...]
    m = jnp.max(z, axis=1, keepdims=True)
    z = z - m
    lse = jnp.log(jnp.sum(jnp.exp(z), axis=1, keepdims=True))
    o_ref[...] = z - lse


def kernel(x, edge_index, W1, b1, W2, b2):
    src = edge_index[0].astype(jnp.int32).reshape(NW, NB, BATCH)
    dst = edge_index[1].astype(jnp.int32).reshape(NW, NB, BATCH)
    zrow = jnp.zeros((ROWS_PER_TILE, NHID), jnp.float32)

    support1 = pl.pallas_call(
        _mm1_body,
        out_shape=jax.ShapeDtypeStruct((N_NODES, NHID), jnp.float32),
    )(x, W1)

    part1 = _aggregate(support1, src, dst, zrow)

    h = pl.pallas_call(
        _relu_body,
        out_shape=jax.ShapeDtypeStruct((N_PAD, NHID), jnp.float32),
    )(part1, b1.reshape(1, NHID))

    part2 = _aggregate(h, src, dst, zrow)

    out = pl.pallas_call(
        _head_body,
        out_shape=jax.ShapeDtypeStruct((N_NODES, NUM_CLASS), jnp.float32),
    )(part2, W2, b2.reshape(1, NUM_CLASS))

    return out


# speedup vs baseline: 23.7259x; 1.1267x over previous
import functools

import jax
import jax.numpy as jnp
from jax import lax
from jax.experimental import pallas as pl
from jax.experimental.pallas import tpu as pltpu
from jax.experimental.pallas import tpu_sc as plsc

N_NODES = 10000
N_EDGES = 320000
FEAT_DIM = 128
NHID = 16
NUM_CLASS = 40

NC = 2
NS = 16
NW = NC * NS
E_PER_W = N_EDGES // NW
BATCH = 125
NB = E_PER_W // BATCH
N_PAD = 10240
ROWS_PER_TILE = N_PAD // NS


GB = 8
NCH = NB // GB


def _agg_body(sup_hbm, edges_hbm, zrow_hbm, out_hbm,
              src_v, dst_v, rows_v, drain_v, acc_sh, gsem, ssem):
    c = lax.axis_index("c")
    s = lax.axis_index("s")
    wid = c * NS + s
    pltpu.sync_copy(edges_hbm.at[0, wid], src_v)
    pltpu.sync_copy(edges_hbm.at[1, wid], dst_v)
    pltpu.sync_copy(zrow_hbm, drain_v)
    pltpu.sync_copy(drain_v, acc_sh.at[pl.ds(s * ROWS_PER_TILE, ROWS_PER_TILE)])
    plsc.subcore_barrier()

    def gather(q, b):
        pltpu.async_copy(sup_hbm.at[src_v.at[q]], rows_v.at[b], gsem.at[b])

    def gather_wait(q, b):
        pltpu.make_async_copy(sup_hbm.at[src_v.at[q]], rows_v.at[b],
                              gsem.at[b]).wait()

    def scat(q, b):
        pltpu.async_copy(rows_v.at[b], acc_sh.at[dst_v.at[q]], ssem.at[b],
                         add=True)

    def scat_wait(q, b):
        pltpu.make_async_copy(rows_v.at[b], acc_sh.at[dst_v.at[q]],
                              ssem.at[b]).wait()

    gather(0, 0)

    def step(q, carry):
        b = lax.rem(q, 2)
        gather_wait(q, b)

        @pl.when(q >= 1)
        def _():
            scat_wait(q - 1, 1 - b)

        @pl.when(q + 1 < NCH)
        def _():
            gather(q + 1, 1 - b)

        scat(q, b)
        return carry

    lax.fori_loop(0, NCH, step, 0)
    scat_wait(NCH - 1, (NCH - 1) % 2)
    plsc.subcore_barrier()
    pltpu.sync_copy(acc_sh.at[pl.ds(s * ROWS_PER_TILE, ROWS_PER_TILE)], drain_v)
    pltpu.sync_copy(drain_v, out_hbm.at[c, pl.ds(s * ROWS_PER_TILE, ROWS_PER_TILE)])


_aggregate = functools.partial(
    pl.kernel,
    mesh=plsc.VectorSubcoreMesh(core_axis_name="c", subcore_axis_name="s"),
    out_type=jax.ShapeDtypeStruct((NC, N_PAD, NHID), jnp.float32),
    scratch_types=[
        pltpu.VMEM((NCH, GB * BATCH), jnp.int32),
        pltpu.VMEM((NCH, GB * BATCH), jnp.int32),
        pltpu.VMEM((2, GB * BATCH, NHID), jnp.float32),
        pltpu.VMEM((ROWS_PER_TILE, NHID), jnp.float32),
        pltpu.VMEM_SHARED((N_PAD, NHID), jnp.float32),
        pltpu.SemaphoreType.DMA((2,)),
        pltpu.SemaphoreType.DMA((2,)),
    ],
    compiler_params=pltpu.CompilerParams(use_tc_tiling_on_sc=False),
)(_agg_body)


def _mm1_body(x_ref, w_ref, o_ref):
    o_ref[...] = jnp.dot(x_ref[...], w_ref[...],
                         preferred_element_type=jnp.float32)


def _relu_body(p_ref, b_ref, o_ref):
    o_ref[...] = jnp.maximum(p_ref[0] + p_ref[1] + b_ref[...], 0.0)


def _head_body(p_ref, w_ref, b_ref, o_ref):
    z = jnp.dot(p_ref[0, :N_NODES] + p_ref[1, :N_NODES], w_ref[...],
                preferred_element_type=jnp.float32) + b_ref[...]
    m = jnp.max(z, axis=1, keepdims=True)
    z = z - m
    lse = jnp.log(jnp.sum(jnp.exp(z), axis=1, keepdims=True))
    o_ref[...] = z - lse


def kernel(x, edge_index, W1, b1, W2, b2):
    edges = edge_index.astype(jnp.int32).reshape(2, NW, NCH, GB * BATCH)
    zrow = jnp.zeros((ROWS_PER_TILE, NHID), jnp.float32)

    support1 = pl.pallas_call(
        _mm1_body,
        out_shape=jax.ShapeDtypeStruct((N_NODES, NHID), jnp.float32),
    )(x, W1)

    part1 = _aggregate(support1, edges, zrow)

    h = pl.pallas_call(
        _relu_body,
        out_shape=jax.ShapeDtypeStruct((N_PAD, NHID), jnp.float32),
    )(part1, b1.reshape(1, NHID))

    part2 = _aggregate(h, edges, zrow)

    out = pl.pallas_call(
        _head_body,
        out_shape=jax.ShapeDtypeStruct((N_NODES, NUM_CLASS), jnp.float32),
    )(part2, W2, b2.reshape(1, NUM_CLASS))

    return out


# speedup vs baseline: 25.5741x; 1.0779x over previous
import functools

import jax
import jax.numpy as jnp
from jax import lax
from jax.experimental import pallas as pl
from jax.experimental.pallas import tpu as pltpu
from jax.experimental.pallas import tpu_sc as plsc

N_NODES = 10000
N_EDGES = 320000
FEAT_DIM = 128
NHID = 16
NUM_CLASS = 40

NC = 2
NS = 16
NW = NC * NS
E_PER_W = N_EDGES // NW
BATCH = 125
NB = E_PER_W // BATCH
N_PAD = 10240
ROWS_PER_TILE = N_PAD // NS
SUP_PER_TILE = N_NODES // NS


GB = 8
NCH = NB // GB


def _agg_body(sup_hbm, edges_hbm, zrow_hbm, out_hbm,
              src_v, dst_v, rows_v, drain_v, acc_sh, sup_sh, gsem, ssem):
    c = lax.axis_index("c")
    s = lax.axis_index("s")
    wid = c * NS + s
    pltpu.sync_copy(edges_hbm.at[0, wid], src_v)
    pltpu.sync_copy(edges_hbm.at[1, wid], dst_v)
    pltpu.sync_copy(sup_hbm.at[pl.ds(s * SUP_PER_TILE, SUP_PER_TILE)],
                    sup_sh.at[pl.ds(s * SUP_PER_TILE, SUP_PER_TILE)])
    pltpu.sync_copy(zrow_hbm, drain_v)
    pltpu.sync_copy(drain_v, acc_sh.at[pl.ds(s * ROWS_PER_TILE, ROWS_PER_TILE)])
    plsc.subcore_barrier()

    def gather(q, b):
        pltpu.async_copy(sup_sh.at[src_v.at[q]], rows_v.at[b], gsem.at[b])

    def gather_wait(q, b):
        pltpu.make_async_copy(sup_sh.at[src_v.at[q]], rows_v.at[b],
                              gsem.at[b]).wait()

    def scat(q, b):
        pltpu.async_copy(rows_v.at[b], acc_sh.at[dst_v.at[q]], ssem.at[b],
                         add=True)

    def scat_wait(q, b):
        pltpu.make_async_copy(rows_v.at[b], acc_sh.at[dst_v.at[q]],
                              ssem.at[b]).wait()

    gather(0, 0)

    def step(q, carry):
        b = lax.rem(q, 2)
        gather_wait(q, b)

        @pl.when(q >= 1)
        def _():
            scat_wait(q - 1, 1 - b)

        @pl.when(q + 1 < NCH)
        def _():
            gather(q + 1, 1 - b)

        scat(q, b)
        return carry

    lax.fori_loop(0, NCH, step, 0)
    scat_wait(NCH - 1, (NCH - 1) % 2)
    plsc.subcore_barrier()
    pltpu.sync_copy(acc_sh.at[pl.ds(s * ROWS_PER_TILE, ROWS_PER_TILE)], drain_v)
    pltpu.sync_copy(drain_v, out_hbm.at[c, pl.ds(s * ROWS_PER_TILE, ROWS_PER_TILE)])


_aggregate = functools.partial(
    pl.kernel,
    mesh=plsc.VectorSubcoreMesh(core_axis_name="c", subcore_axis_name="s"),
    out_type=jax.ShapeDtypeStruct((NC, N_PAD, NHID), jnp.float32),
    scratch_types=[
        pltpu.VMEM((NCH, GB * BATCH), jnp.int32),
        pltpu.VMEM((NCH, GB * BATCH), jnp.int32),
        pltpu.VMEM((2, GB * BATCH, NHID), jnp.float32),
        pltpu.VMEM((ROWS_PER_TILE, NHID), jnp.float32),
        pltpu.VMEM_SHARED((N_PAD, NHID), jnp.float32),
        pltpu.VMEM_SHARED((N_NODES, NHID), jnp.float32),
        pltpu.SemaphoreType.DMA((2,)),
        pltpu.SemaphoreType.DMA((2,)),
    ],
    compiler_params=pltpu.CompilerParams(use_tc_tiling_on_sc=False),
)(_agg_body)


def _mm1_body(x_ref, w_ref, o_ref):
    o_ref[...] = jnp.dot(x_ref[...], w_ref[...],
                         preferred_element_type=jnp.float32)


def _relu_body(p_ref, b_ref, o_ref):
    o_ref[...] = jnp.maximum(p_ref[0] + p_ref[1] + b_ref[...], 0.0)


def _head_body(p_ref, w_ref, b_ref, o_ref):
    z = jnp.dot(p_ref[0, :N_NODES] + p_ref[1, :N_NODES], w_ref[...],
                preferred_element_type=jnp.float32) + b_ref[...]
    m = jnp.max(z, axis=1, keepdims=True)
    z = z - m
    lse = jnp.log(jnp.sum(jnp.exp(z), axis=1, keepdims=True))
    o_ref[...] = z - lse


def kernel(x, edge_index, W1, b1, W2, b2):
    edges = edge_index.astype(jnp.int32).reshape(2, NW, NCH, GB * BATCH)
    zrow = jnp.zeros((ROWS_PER_TILE, NHID), jnp.float32)

    support1 = pl.pallas_call(
        _mm1_body,
        out_shape=jax.ShapeDtypeStruct((N_NODES, NHID), jnp.float32),
    )(x, W1)

    part1 = _aggregate(support1, edges, zrow)

    h = pl.pallas_call(
        _relu_body,
        out_shape=jax.ShapeDtypeStruct((N_PAD, NHID), jnp.float32),
    )(part1, b1.reshape(1, NHID))

    part2 = _aggregate(h, edges, zrow)

    out = pl.pallas_call(
        _head_body,
        out_shape=jax.ShapeDtypeStruct((N_NODES, NUM_CLASS), jnp.float32),
    )(part2, W2, b2.reshape(1, NUM_CLASS))

    return out


# speedup vs baseline: 28.0046x; 1.0950x over previous
import functools

import jax
import jax.numpy as jnp
from jax import lax
from jax.experimental import pallas as pl
from jax.experimental.pallas import tpu as pltpu
from jax.experimental.pallas import tpu_sc as plsc

N_NODES = 10000
N_EDGES = 320000
FEAT_DIM = 128
NHID = 16
NUM_CLASS = 40

NC = 2
NS = 16
NW = NC * NS
E_PER_W = N_EDGES // NW
BATCH = 125
NB = E_PER_W // BATCH
N_PAD = 10240
ROWS_PER_TILE = N_PAD // NS
SUP_PER_TILE = N_NODES // NS


GB = 8
NCH = NB // GB


def _stage_common(edges_hbm, wid, s, src_v, dst_v, drain_v, acc_sh):
    pltpu.sync_copy(edges_hbm.at[0, pl.ds(wid * E_PER_W, E_PER_W)], src_v)
    pltpu.sync_copy(edges_hbm.at[1, pl.ds(wid * E_PER_W, E_PER_W)], dst_v)
    zero16 = jnp.zeros((16,), jnp.float32)

    def zrow(i, carry):
        drain_v[i] = zero16
        return carry

    lax.fori_loop(0, ROWS_PER_TILE, zrow, 0)
    pltpu.sync_copy(drain_v, acc_sh.at[pl.ds(s * ROWS_PER_TILE, ROWS_PER_TILE)])


def _agg_loop(s, c, src_v, dst_v, rows_v, drain_v, acc_sh, sup_sh, gsem, ssem,
              out_hbm):
    CH = GB * BATCH

    def gather(q, b):
        pltpu.async_copy(sup_sh.at[src_v.at[pl.ds(q * CH, CH)]], rows_v.at[b],
                         gsem.at[b])

    def gather_wait(q, b):
        pltpu.make_async_copy(sup_sh.at[src_v.at[pl.ds(q * CH, CH)]],
                              rows_v.at[b], gsem.at[b]).wait()

    def scat(q, b):
        pltpu.async_copy(rows_v.at[b], acc_sh.at[dst_v.at[pl.ds(q * CH, CH)]],
                         ssem.at[b], add=True)

    def scat_wait(q, b):
        pltpu.make_async_copy(rows_v.at[b],
                              acc_sh.at[dst_v.at[pl.ds(q * CH, CH)]],
                              ssem.at[b]).wait()

    gather(0, 0)

    def step(q, carry):
        b = lax.rem(q, 2)
        gather_wait(q, b)

        @pl.when(q >= 1)
        def _():
            scat_wait(q - 1, 1 - b)

        @pl.when(q + 1 < NCH)
        def _():
            gather(q + 1, 1 - b)

        scat(q, b)
        return carry

    lax.fori_loop(0, NCH, step, 0)
    scat_wait(NCH - 1, (NCH - 1) % 2)
    plsc.subcore_barrier()
    pltpu.sync_copy(acc_sh.at[pl.ds(s * ROWS_PER_TILE, ROWS_PER_TILE)], drain_v)
    pltpu.sync_copy(drain_v, out_hbm.at[c, pl.ds(s * ROWS_PER_TILE, ROWS_PER_TILE)])


def _agg1_body(sup_hbm, edges_hbm, out_hbm,
               src_v, dst_v, rows_v, drain_v, acc_sh, sup_sh, gsem, ssem):
    c = lax.axis_index("c")
    s = lax.axis_index("s")
    wid = c * NS + s
    _stage_common(edges_hbm, wid, s, src_v, dst_v, drain_v, acc_sh)
    pltpu.sync_copy(sup_hbm.at[pl.ds(s * SUP_PER_TILE, SUP_PER_TILE)],
                    sup_sh.at[pl.ds(s * SUP_PER_TILE, SUP_PER_TILE)])
    plsc.subcore_barrier()
    _agg_loop(s, c, src_v, dst_v, rows_v, drain_v, acc_sh, sup_sh, gsem, ssem,
              out_hbm)


def _agg2_body(part_hbm, edges_hbm, b_hbm, out_hbm,
               src_v, dst_v, rows_v, drain_v, acc_sh, sup_sh, p0_v, p1_v, b_v,
               gsem, ssem):
    c = lax.axis_index("c")
    s = lax.axis_index("s")
    wid = c * NS + s
    _stage_common(edges_hbm, wid, s, src_v, dst_v, drain_v, acc_sh)
    pltpu.sync_copy(part_hbm.at[0, pl.ds(s * SUP_PER_TILE, SUP_PER_TILE)], p0_v)
    pltpu.sync_copy(part_hbm.at[1, pl.ds(s * SUP_PER_TILE, SUP_PER_TILE)], p1_v)
    pltpu.sync_copy(b_hbm, b_v)
    bvec = b_v[...]

    def row(i, carry):
        p0_v[i] = jnp.maximum(p0_v[i] + p1_v[i] + bvec, 0.0)
        return carry

    lax.fori_loop(0, SUP_PER_TILE, row, 0)
    pltpu.sync_copy(p0_v, sup_sh.at[pl.ds(s * SUP_PER_TILE, SUP_PER_TILE)])
    plsc.subcore_barrier()
    _agg_loop(s, c, src_v, dst_v, rows_v, drain_v, acc_sh, sup_sh, gsem, ssem,
              out_hbm)


_SCRATCH_COMMON = [
    pltpu.VMEM((E_PER_W,), jnp.int32),
    pltpu.VMEM((E_PER_W,), jnp.int32),
    pltpu.VMEM((2, GB * BATCH, NHID), jnp.float32),
    pltpu.VMEM((ROWS_PER_TILE, NHID), jnp.float32),
    pltpu.VMEM_SHARED((N_PAD, NHID), jnp.float32),
    pltpu.VMEM_SHARED((N_NODES, NHID), jnp.float32),
]
_SCRATCH_SEMS = [
    pltpu.SemaphoreType.DMA((2,)),
    pltpu.SemaphoreType.DMA((2,)),
]

_aggregate1 = functools.partial(
    pl.kernel,
    mesh=plsc.VectorSubcoreMesh(core_axis_name="c", subcore_axis_name="s"),
    out_type=jax.ShapeDtypeStruct((NC, N_PAD, NHID), jnp.float32),
    scratch_types=_SCRATCH_COMMON + _SCRATCH_SEMS,
    compiler_params=pltpu.CompilerParams(use_tc_tiling_on_sc=False),
)(_agg1_body)

_aggregate2 = functools.partial(
    pl.kernel,
    mesh=plsc.VectorSubcoreMesh(core_axis_name="c", subcore_axis_name="s"),
    out_type=jax.ShapeDtypeStruct((NC, N_PAD, NHID), jnp.float32),
    scratch_types=_SCRATCH_COMMON + [
        pltpu.VMEM((SUP_PER_TILE, NHID), jnp.float32),
        pltpu.VMEM((SUP_PER_TILE, NHID), jnp.float32),
        pltpu.VMEM((NHID,), jnp.float32),
    ] + _SCRATCH_SEMS,
    compiler_params=pltpu.CompilerParams(use_tc_tiling_on_sc=False),
)(_agg2_body)


def _mm1_body(x_ref, w_ref, o_ref):
    o_ref[...] = jnp.dot(x_ref[...], w_ref[...],
                         preferred_element_type=jnp.float32)


def _head_body(p_ref, w_ref, b_ref, o_ref):
    z = jnp.dot(p_ref[0, :N_NODES] + p_ref[1, :N_NODES], w_ref[...],
                preferred_element_type=jnp.float32) + b_ref[...]
    m = jnp.max(z, axis=1, keepdims=True)
    z = z - m
    lse = jnp.log(jnp.sum(jnp.exp(z), axis=1, keepdims=True))
    o_ref[...] = z - lse


def kernel(x, edge_index, W1, b1, W2, b2):
    edges = edge_index.astype(jnp.int32)

    support1 = pl.pallas_call(
        _mm1_body,
        out_shape=jax.ShapeDtypeStruct((N_NODES, NHID), jnp.float32),
    )(x, W1)

    part1 = _aggregate1(support1, edges)
    part2 = _aggregate2(part1, edges, b1)

    out = pl.pallas_call(
        _head_body,
        out_shape=jax.ShapeDtypeStruct((N_NODES, NUM_CLASS), jnp.float32),
    )(part2, W2, b2.reshape(1, NUM_CLASS))

    return out


# speedup vs baseline: 30.5392x; 1.0905x over previous
import functools

import jax
import jax.numpy as jnp
from jax import lax
from jax.experimental import pallas as pl
from jax.experimental.pallas import tpu as pltpu
from jax.experimental.pallas import tpu_sc as plsc

N_NODES = 10000
N_EDGES = 320000
FEAT_DIM = 128
NHID = 16
NUM_CLASS = 40

NC = 2
NS = 16
NW = NC * NS
E_PER_W = N_EDGES // NW
BATCH = 125
NB = E_PER_W // BATCH
N_PAD = 10240
ROWS_PER_TILE = N_PAD // NS
SUP_PER_TILE = N_NODES // NS


GB = 8
NCH = NB // GB


def _stage_common(edges_hbm, wid, s, src_v, dst_v, drain_v, acc_sh):
    pltpu.sync_copy(edges_hbm.at[0, pl.ds(wid * E_PER_W, E_PER_W)], src_v)
    pltpu.sync_copy(edges_hbm.at[1, pl.ds(wid * E_PER_W, E_PER_W)], dst_v)
    zero16 = jnp.zeros((16,), jnp.float32)

    def zrow(i, carry):
        drain_v[i] = zero16
        return carry

    lax.fori_loop(0, ROWS_PER_TILE, zrow, 0)
    pltpu.sync_copy(drain_v, acc_sh.at[pl.ds(s * ROWS_PER_TILE, ROWS_PER_TILE)])


def _agg_loop(s, c, src_v, dst_v, rows_v, drain_v, acc_sh, sup_sh, gsem, ssem,
              out_hbm):
    CH = GB * BATCH

    def gather(q, b):
        pltpu.async_copy(sup_sh.at[src_v.at[pl.ds(q * CH, CH)]], rows_v.at[b],
                         gsem.at[b])

    def gather_wait(q, b):
        pltpu.make_async_copy(sup_sh.at[src_v.at[pl.ds(q * CH, CH)]],
                              rows_v.at[b], gsem.at[b]).wait()

    def scat(q, b):
        pltpu.async_copy(rows_v.at[b], acc_sh.at[dst_v.at[pl.ds(q * CH, CH)]],
                         ssem.at[b], add=True)

    def scat_wait(q, b):
        pltpu.make_async_copy(rows_v.at[b],
                              acc_sh.at[dst_v.at[pl.ds(q * CH, CH)]],
                              ssem.at[b]).wait()

    gather(0, 0)

    def step(q, carry):
        b = lax.rem(q, 2)
        gather_wait(q, b)

        @pl.when(q >= 1)
        def _():
            scat_wait(q - 1, 1 - b)

        @pl.when(q + 1 < NCH)
        def _():
            gather(q + 1, 1 - b)

        scat(q, b)
        return carry

    lax.fori_loop(0, NCH, step, 0)
    scat_wait(NCH - 1, (NCH - 1) % 2)
    plsc.subcore_barrier()
    pltpu.sync_copy(acc_sh.at[pl.ds(s * ROWS_PER_TILE, ROWS_PER_TILE)], drain_v)
    if out_hbm.shape[2] == NHID:
        pltpu.sync_copy(drain_v,
                        out_hbm.at[c, pl.ds(s * ROWS_PER_TILE, ROWS_PER_TILE)])
    else:
        pltpu.sync_copy(drain_v,
                        out_hbm.at[c, pl.ds(s * ROWS_PER_TILE, ROWS_PER_TILE),
                                   pl.ds(0, NHID)])


def _agg1_body(sup_hbm, edges_hbm, out_hbm,
               src_v, dst_v, rows_v, drain_v, acc_sh, sup_sh, gsem, ssem):
    c = lax.axis_index("c")
    s = lax.axis_index("s")
    wid = c * NS + s
    _stage_common(edges_hbm, wid, s, src_v, dst_v, drain_v, acc_sh)
    pltpu.sync_copy(sup_hbm.at[pl.ds(s * SUP_PER_TILE, SUP_PER_TILE),
                               pl.ds(0, NHID)],
                    sup_sh.at[pl.ds(s * SUP_PER_TILE, SUP_PER_TILE)])
    plsc.subcore_barrier()
    _agg_loop(s, c, src_v, dst_v, rows_v, drain_v, acc_sh, sup_sh, gsem, ssem,
              out_hbm)


def _agg2_body(part_hbm, edges_hbm, b_hbm, out_hbm,
               src_v, dst_v, rows_v, drain_v, acc_sh, sup_sh, p0_v, p1_v, b_v,
               gsem, ssem):
    c = lax.axis_index("c")
    s = lax.axis_index("s")
    wid = c * NS + s
    _stage_common(edges_hbm, wid, s, src_v, dst_v, drain_v, acc_sh)
    pltpu.sync_copy(part_hbm.at[0, pl.ds(s * SUP_PER_TILE, SUP_PER_TILE)], p0_v)
    pltpu.sync_copy(part_hbm.at[1, pl.ds(s * SUP_PER_TILE, SUP_PER_TILE)], p1_v)
    pltpu.sync_copy(b_hbm, b_v)
    bvec = b_v[...]

    def row(i, carry):
        p0_v[i] = jnp.maximum(p0_v[i] + p1_v[i] + bvec, 0.0)
        return carry

    lax.fori_loop(0, SUP_PER_TILE, row, 0)
    pltpu.sync_copy(p0_v, sup_sh.at[pl.ds(s * SUP_PER_TILE, SUP_PER_TILE)])
    plsc.subcore_barrier()
    _agg_loop(s, c, src_v, dst_v, rows_v, drain_v, acc_sh, sup_sh, gsem, ssem,
              out_hbm)


_SCRATCH_COMMON = [
    pltpu.VMEM((E_PER_W,), jnp.int32),
    pltpu.VMEM((E_PER_W,), jnp.int32),
    pltpu.VMEM((2, GB * BATCH, NHID), jnp.float32),
    pltpu.VMEM((ROWS_PER_TILE, NHID), jnp.float32),
    pltpu.VMEM_SHARED((N_PAD, NHID), jnp.float32),
    pltpu.VMEM_SHARED((N_NODES, NHID), jnp.float32),
]
_SCRATCH_SEMS = [
    pltpu.SemaphoreType.DMA((2,)),
    pltpu.SemaphoreType.DMA((2,)),
]

_aggregate1 = functools.partial(
    pl.kernel,
    mesh=plsc.VectorSubcoreMesh(core_axis_name="c", subcore_axis_name="s"),
    out_type=jax.ShapeDtypeStruct((NC, N_PAD, NHID), jnp.float32),
    scratch_types=_SCRATCH_COMMON + _SCRATCH_SEMS,
    compiler_params=pltpu.CompilerParams(use_tc_tiling_on_sc=False),
)(_agg1_body)

_aggregate2 = functools.partial(
    pl.kernel,
    mesh=plsc.VectorSubcoreMesh(core_axis_name="c", subcore_axis_name="s"),
    out_type=jax.ShapeDtypeStruct((NC, N_PAD, 128), jnp.float32),
    scratch_types=_SCRATCH_COMMON + [
        pltpu.VMEM((SUP_PER_TILE, NHID), jnp.float32),
        pltpu.VMEM((SUP_PER_TILE, NHID), jnp.float32),
        pltpu.VMEM((NHID,), jnp.float32),
    ] + _SCRATCH_SEMS,
    compiler_params=pltpu.CompilerParams(use_tc_tiling_on_sc=False),
)(_agg2_body)


def _mm1_body(x_ref, w_ref, o_ref):
    o_ref[:, :NHID] = jnp.dot(x_ref[...], w_ref[...],
                              preferred_element_type=jnp.float32)


def _head_body(p_ref, w_ref, b_ref, o_ref):
    p = p_ref[0, :N_NODES, :NHID] + p_ref[1, :N_NODES, :NHID]
    z = jnp.dot(p, w_ref[...], preferred_element_type=jnp.float32) + b_ref[...]
    m = jnp.max(z, axis=1, keepdims=True)
    z = z - m
    lse = jnp.log(jnp.sum(jnp.exp(z), axis=1, keepdims=True))
    o_ref[...] = z - lse


def kernel(x, edge_index, W1, b1, W2, b2):
    edges = edge_index.astype(jnp.int32)

    support1 = pl.pallas_call(
        _mm1_body,
        out_shape=jax.ShapeDtypeStruct((N_NODES, 128), jnp.float32),
    )(x, W1)

    part1 = _aggregate1(support1, edges)
    part2 = _aggregate2(part1, edges, b1)

    out = pl.pallas_call(
        _head_body,
        out_shape=jax.ShapeDtypeStruct((N_NODES, NUM_CLASS), jnp.float32),
    )(part2, W2, b2.reshape(1, NUM_CLASS))

    return out


# speedup vs baseline: 33.2693x; 1.0894x over previous
import functools

import jax
import jax.numpy as jnp
from jax import lax
from jax.experimental import pallas as pl
from jax.experimental.pallas import tpu as pltpu
from jax.experimental.pallas import tpu_sc as plsc

N_NODES = 10000
N_EDGES = 320000
FEAT_DIM = 128
NHID = 16
NUM_CLASS = 40

NC = 2
NS = 16
NW = NC * NS
E_PER_W = N_EDGES // NW
BATCH = 125
NB = E_PER_W // BATCH
N_PAD = 10240
ROWS_PER_TILE = N_PAD // NS
SUP_PER_TILE = N_NODES // NS


GB = 8
NCH = NB // GB


def _stage_common(edges_hbm, wid, s, src_v, dst_v, drain_v, acc_sh):
    pltpu.sync_copy(edges_hbm.at[0, pl.ds(wid * E_PER_W, E_PER_W)], src_v)
    pltpu.sync_copy(edges_hbm.at[1, pl.ds(wid * E_PER_W, E_PER_W)], dst_v)
    zero16 = jnp.zeros((16,), jnp.float32)

    def zrow(i, carry):
        for u in range(8):
            drain_v[i * 8 + u] = zero16
        return carry

    lax.fori_loop(0, ROWS_PER_TILE // 8, zrow, 0)
    pltpu.sync_copy(drain_v, acc_sh.at[pl.ds(s * ROWS_PER_TILE, ROWS_PER_TILE)])


def _agg_loop(s, c, src_v, dst_v, rows_v, drain_v, acc_sh, sup_sh, gsem, ssem,
              out_hbm):
    CH = GB * BATCH

    def gather(q, b):
        pltpu.async_copy(sup_sh.at[src_v.at[pl.ds(q * CH, CH)]], rows_v.at[b],
                         gsem.at[b])

    def gather_wait(q, b):
        pltpu.make_async_copy(sup_sh.at[src_v.at[pl.ds(q * CH, CH)]],
                              rows_v.at[b], gsem.at[b]).wait()

    def scat(q, b):
        pltpu.async_copy(rows_v.at[b], acc_sh.at[dst_v.at[pl.ds(q * CH, CH)]],
                         ssem.at[b], add=True)

    def scat_wait(q, b):
        pltpu.make_async_copy(rows_v.at[b],
                              acc_sh.at[dst_v.at[pl.ds(q * CH, CH)]],
                              ssem.at[b]).wait()

    gather(0, 0)

    def step(q, carry):
        b = lax.rem(q, 2)
        gather_wait(q, b)

        @pl.when(q >= 1)
        def _():
            scat_wait(q - 1, 1 - b)

        @pl.when(q + 1 < NCH)
        def _():
            gather(q + 1, 1 - b)

        scat(q, b)
        return carry

    lax.fori_loop(0, NCH, step, 0)
    scat_wait(NCH - 1, (NCH - 1) % 2)
    plsc.subcore_barrier()
    pltpu.sync_copy(acc_sh.at[pl.ds(s * ROWS_PER_TILE, ROWS_PER_TILE)], drain_v)
    if out_hbm.shape[2] == NHID:
        pltpu.sync_copy(drain_v,
                        out_hbm.at[c, pl.ds(s * ROWS_PER_TILE, ROWS_PER_TILE)])
    else:
        pltpu.sync_copy(drain_v,
                        out_hbm.at[c, pl.ds(s * ROWS_PER_TILE, ROWS_PER_TILE),
                                   pl.ds(0, NHID)])


def _agg1_body(sup_hbm, edges_hbm, out_hbm,
               src_v, dst_v, rows_v, drain_v, acc_sh, sup_sh, gsem, ssem):
    c = lax.axis_index("c")
    s = lax.axis_index("s")
    wid = c * NS + s
    _stage_common(edges_hbm, wid, s, src_v, dst_v, drain_v, acc_sh)
    pltpu.sync_copy(sup_hbm.at[pl.ds(s * SUP_PER_TILE, SUP_PER_TILE),
                               pl.ds(0, NHID)],
                    sup_sh.at[pl.ds(s * SUP_PER_TILE, SUP_PER_TILE)])
    plsc.subcore_barrier()
    _agg_loop(s, c, src_v, dst_v, rows_v, drain_v, acc_sh, sup_sh, gsem, ssem,
              out_hbm)


def _agg2_body(part_hbm, edges_hbm, b_hbm, out_hbm,
               src_v, dst_v, rows_v, drain_v, acc_sh, sup_sh, p0_v, p1_v, b_v,
               gsem, ssem):
    c = lax.axis_index("c")
    s = lax.axis_index("s")
    wid = c * NS + s
    _stage_common(edges_hbm, wid, s, src_v, dst_v, drain_v, acc_sh)
    pltpu.sync_copy(part_hbm.at[0, pl.ds(s * SUP_PER_TILE, SUP_PER_TILE)], p0_v)
    pltpu.sync_copy(part_hbm.at[1, pl.ds(s * SUP_PER_TILE, SUP_PER_TILE)], p1_v)
    pltpu.sync_copy(b_hbm, b_v)
    bvec = b_v[...]

    def row(i, carry):
        for u in range(5):
            j = i * 5 + u
            p0_v[j] = jnp.maximum(p0_v[j] + p1_v[j] + bvec, 0.0)
        return carry

    lax.fori_loop(0, SUP_PER_TILE // 5, row, 0)
    pltpu.sync_copy(p0_v, sup_sh.at[pl.ds(s * SUP_PER_TILE, SUP_PER_TILE)])
    plsc.subcore_barrier()
    _agg_loop(s, c, src_v, dst_v, rows_v, drain_v, acc_sh, sup_sh, gsem, ssem,
              out_hbm)


_SCRATCH_COMMON = [
    pltpu.VMEM((E_PER_W,), jnp.int32),
    pltpu.VMEM((E_PER_W,), jnp.int32),
    pltpu.VMEM((2, GB * BATCH, NHID), jnp.float32),
    pltpu.VMEM((ROWS_PER_TILE, NHID), jnp.float32),
    pltpu.VMEM_SHARED((N_PAD, NHID), jnp.float32),
    pltpu.VMEM_SHARED((N_NODES, NHID), jnp.float32),
]
_SCRATCH_SEMS = [
    pltpu.SemaphoreType.DMA((2,)),
    pltpu.SemaphoreType.DMA((2,)),
]

_aggregate1 = functools.partial(
    pl.kernel,
    mesh=plsc.VectorSubcoreMesh(core_axis_name="c", subcore_axis_name="s"),
    out_type=jax.ShapeDtypeStruct((NC, N_PAD, NHID), jnp.float32),
    scratch_types=_SCRATCH_COMMON + _SCRATCH_SEMS,
    compiler_params=pltpu.CompilerParams(use_tc_tiling_on_sc=False),
)(_agg1_body)

_aggregate2 = functools.partial(
    pl.kernel,
    mesh=plsc.VectorSubcoreMesh(core_axis_name="c", subcore_axis_name="s"),
    out_type=jax.ShapeDtypeStruct((NC, N_PAD, 128), jnp.float32),
    scratch_types=_SCRATCH_COMMON + [
        pltpu.VMEM((SUP_PER_TILE, NHID), jnp.float32),
        pltpu.VMEM((SUP_PER_TILE, NHID), jnp.float32),
        pltpu.VMEM((NHID,), jnp.float32),
    ] + _SCRATCH_SEMS,
    compiler_params=pltpu.CompilerParams(use_tc_tiling_on_sc=False),
)(_agg2_body)


def _mm1_body(x_ref, w_ref, o_ref):
    o_ref[:, :NHID] = jnp.dot(x_ref[...], w_ref[...],
                              preferred_element_type=jnp.float32)


def _head_body(p_ref, w_ref, b_ref, o_ref):
    p = p_ref[0, :N_NODES, :NHID] + p_ref[1, :N_NODES, :NHID]
    z = jnp.dot(p, w_ref[...], preferred_element_type=jnp.float32) + b_ref[...]
    m = jnp.max(z, axis=1, keepdims=True)
    z = z - m
    lse = jnp.log(jnp.sum(jnp.exp(z), axis=1, keepdims=True))
    o_ref[...] = z - lse


def kernel(x, edge_index, W1, b1, W2, b2):
    edges = edge_index.astype(jnp.int32)

    support1 = pl.pallas_call(
        _mm1_body,
        out_shape=jax.ShapeDtypeStruct((N_NODES, 128), jnp.float32),
    )(x, W1)

    part1 = _aggregate1(support1, edges)
    part2 = _aggregate2(part1, edges, b1)

    out = pl.pallas_call(
        _head_body,
        out_shape=jax.ShapeDtypeStruct((N_NODES, NUM_CLASS), jnp.float32),
    )(part2, W2, b2.reshape(1, NUM_CLASS))

    return out


# speedup vs baseline: 33.2739x; 1.0001x over previous
import functools

import jax
import jax.numpy as jnp
from jax import lax
from jax.experimental import pallas as pl
from jax.experimental.pallas import tpu as pltpu
from jax.experimental.pallas import tpu_sc as plsc

N_NODES = 10000
N_EDGES = 320000
FEAT_DIM = 128
NHID = 16
NUM_CLASS = 40

NC = 2
NS = 16
NW = NC * NS
E_PER_W = N_EDGES // NW
BATCH = 125
NB = E_PER_W // BATCH
N_PAD = 10240
ROWS_PER_TILE = N_PAD // NS
SUP_PER_TILE = N_NODES // NS


GB = 8
NCH = NB // GB


def _stage_common(edges_hbm, wid, s, src_v, dst_v, drain_v, acc_sh):
    pltpu.sync_copy(edges_hbm.at[0, pl.ds(wid * E_PER_W, E_PER_W)], src_v)
    pltpu.sync_copy(edges_hbm.at[1, pl.ds(wid * E_PER_W, E_PER_W)], dst_v)
    zero16 = jnp.zeros((16,), jnp.float32)

    def zrow(i, carry):
        for u in range(16):
            drain_v[i * 16 + u] = zero16
        return carry

    lax.fori_loop(0, ROWS_PER_TILE // 16, zrow, 0)
    pltpu.sync_copy(drain_v, acc_sh.at[pl.ds(s * ROWS_PER_TILE, ROWS_PER_TILE)])


def _agg_loop(s, c, src_v, dst_v, rows_v, drain_v, acc_sh, sup_sh, gsem, ssem,
              out_hbm):
    CH = GB * BATCH

    def gather(q, b):
        pltpu.async_copy(sup_sh.at[src_v.at[pl.ds(q * CH, CH)]], rows_v.at[b],
                         gsem.at[b])

    def gather_wait(q, b):
        pltpu.make_async_copy(sup_sh.at[src_v.at[pl.ds(q * CH, CH)]],
                              rows_v.at[b], gsem.at[b]).wait()

    def scat(q, b):
        pltpu.async_copy(rows_v.at[b], acc_sh.at[dst_v.at[pl.ds(q * CH, CH)]],
                         ssem.at[b], add=True)

    def scat_wait(q, b):
        pltpu.make_async_copy(rows_v.at[b],
                              acc_sh.at[dst_v.at[pl.ds(q * CH, CH)]],
                              ssem.at[b]).wait()

    gather(0, 0)

    def step(q, carry):
        b = lax.rem(q, 2)
        gather_wait(q, b)

        @pl.when(q >= 1)
        def _():
            scat_wait(q - 1, 1 - b)

        @pl.when(q + 1 < NCH)
        def _():
            gather(q + 1, 1 - b)

        scat(q, b)
        return carry

    lax.fori_loop(0, NCH, step, 0)
    scat_wait(NCH - 1, (NCH - 1) % 2)
    plsc.subcore_barrier()
    pltpu.sync_copy(acc_sh.at[pl.ds(s * ROWS_PER_TILE, ROWS_PER_TILE)], drain_v)
    if out_hbm.shape[2] == NHID:
        pltpu.sync_copy(drain_v,
                        out_hbm.at[c, pl.ds(s * ROWS_PER_TILE, ROWS_PER_TILE)])
    else:
        pltpu.sync_copy(drain_v,
                        out_hbm.at[c, pl.ds(s * ROWS_PER_TILE, ROWS_PER_TILE),
                                   pl.ds(0, NHID)])


def _agg1_body(sup_hbm, edges_hbm, out_hbm,
               src_v, dst_v, rows_v, drain_v, acc_sh, sup_sh, gsem, ssem):
    c = lax.axis_index("c")
    s = lax.axis_index("s")
    wid = c * NS + s
    _stage_common(edges_hbm, wid, s, src_v, dst_v, drain_v, acc_sh)
    pltpu.sync_copy(sup_hbm.at[pl.ds(s * SUP_PER_TILE, SUP_PER_TILE),
                               pl.ds(0, NHID)],
                    sup_sh.at[pl.ds(s * SUP_PER_TILE, SUP_PER_TILE)])
    plsc.subcore_barrier()
    _agg_loop(s, c, src_v, dst_v, rows_v, drain_v, acc_sh, sup_sh, gsem, ssem,
              out_hbm)


def _agg2_body(part_hbm, edges_hbm, b_hbm, out_hbm,
               src_v, dst_v, rows_v, drain_v, acc_sh, sup_sh, p0_v, p1_v, b_v,
               gsem, ssem):
    c = lax.axis_index("c")
    s = lax.axis_index("s")
    wid = c * NS + s
    _stage_common(edges_hbm, wid, s, src_v, dst_v, drain_v, acc_sh)
    pltpu.sync_copy(part_hbm.at[0, pl.ds(s * SUP_PER_TILE, SUP_PER_TILE)], p0_v)
    pltpu.sync_copy(part_hbm.at[1, pl.ds(s * SUP_PER_TILE, SUP_PER_TILE)], p1_v)
    pltpu.sync_copy(b_hbm, b_v)
    bvec = b_v[...]

    def row(i, carry):
        for u in range(25):
            j = i * 25 + u
            p0_v[j] = jnp.maximum(p0_v[j] + p1_v[j] + bvec, 0.0)
        return carry

    lax.fori_loop(0, SUP_PER_TILE // 25, row, 0)
    pltpu.sync_copy(p0_v, sup_sh.at[pl.ds(s * SUP_PER_TILE, SUP_PER_TILE)])
    plsc.subcore_barrier()
    _agg_loop(s, c, src_v, dst_v, rows_v, drain_v, acc_sh, sup_sh, gsem, ssem,
              out_hbm)


_SCRATCH_COMMON = [
    pltpu.VMEM((E_PER_W,), jnp.int32),
    pltpu.VMEM((E_PER_W,), jnp.int32),
    pltpu.VMEM((2, GB * BATCH, NHID), jnp.float32),
    pltpu.VMEM((ROWS_PER_TILE, NHID), jnp.float32),
    pltpu.VMEM_SHARED((N_PAD, NHID), jnp.float32),
    pltpu.VMEM_SHARED((N_NODES, NHID), jnp.float32),
]
_SCRATCH_SEMS = [
    pltpu.SemaphoreType.DMA((2,)),
    pltpu.SemaphoreType.DMA((2,)),
]

_aggregate1 = functools.partial(
    pl.kernel,
    mesh=plsc.VectorSubcoreMesh(core_axis_name="c", subcore_axis_name="s"),
    out_type=jax.ShapeDtypeStruct((NC, N_PAD, NHID), jnp.float32),
    scratch_types=_SCRATCH_COMMON + _SCRATCH_SEMS,
    compiler_params=pltpu.CompilerParams(use_tc_tiling_on_sc=False),
)(_agg1_body)

_aggregate2 = functools.partial(
    pl.kernel,
    mesh=plsc.VectorSubcoreMesh(core_axis_name="c", subcore_axis_name="s"),
    out_type=jax.ShapeDtypeStruct((NC, N_PAD, 128), jnp.float32),
    scratch_types=_SCRATCH_COMMON + [
        pltpu.VMEM((SUP_PER_TILE, NHID), jnp.float32),
        pltpu.VMEM((SUP_PER_TILE, NHID), jnp.float32),
        pltpu.VMEM((NHID,), jnp.float32),
    ] + _SCRATCH_SEMS,
    compiler_params=pltpu.CompilerParams(use_tc_tiling_on_sc=False),
)(_agg2_body)


def _mm1_body(x_ref, w_ref, o_ref):
    o_ref[:, :NHID] = jnp.dot(x_ref[...], w_ref[...],
                              preferred_element_type=jnp.float32)


def _head_body(p_ref, w_ref, b_ref, o_ref):
    p = p_ref[0, :N_NODES, :NHID] + p_ref[1, :N_NODES, :NHID]
    z = jnp.dot(p, w_ref[...], preferred_element_type=jnp.float32) + b_ref[...]
    m = jnp.max(z, axis=1, keepdims=True)
    z = z - m
    lse = jnp.log(jnp.sum(jnp.exp(z), axis=1, keepdims=True))
    o_ref[...] = z - lse


def kernel(x, edge_index, W1, b1, W2, b2):
    edges = edge_index.astype(jnp.int32)

    support1 = pl.pallas_call(
        _mm1_body,
        out_shape=jax.ShapeDtypeStruct((N_NODES, 128), jnp.float32),
    )(x, W1)

    part1 = _aggregate1(support1, edges)
    part2 = _aggregate2(part1, edges, b1)

    out = pl.pallas_call(
        _head_body,
        out_shape=jax.ShapeDtypeStruct((N_NODES, NUM_CLASS), jnp.float32),
    )(part2, W2, b2.reshape(1, NUM_CLASS))

    return out
